# baseline (speedup 1.0000x reference)
import jax
import jax.numpy as jnp
from jax import lax
from jax.experimental import pallas as pl
from jax.experimental.pallas import tpu as pltpu
from jax.experimental.pallas import tpu_sc as plsc

L_EV = 2_000_000
CHUNK = 8192
NCH_FULL = L_EV // CHUNK
TAIL = L_EV - NCH_FULL * CHUNK
NW = 32
EXTRA = NCH_FULL - 7 * NW
NQ = 10_000
QPW = 320
QV = QPW // 16
ROWS_C = CHUNK // 128
VPR = 128 // 16
NROWS = L_EV // 128
TAIL_ROWS = TAIL // 128

_mesh = plsc.VectorSubcoreMesh(
    core_axis_name="c", subcore_axis_name="s", num_cores=2, num_subcores=16)


def _stage_a_body(x_hbm, y_hbm, box_hbm, counts_hbm, rowincl_hbm,
                  xbuf, ybuf, rowbuf, boxv, cntv):
    wid = lax.axis_index("c") * 16 + lax.axis_index("s")
    pltpu.sync_copy(box_hbm, boxv)
    c0 = 7 * wid + jnp.minimum(wid, EXTRA)
    nch = jnp.where(wid < EXTRA, 8, 7)
    xlo = boxv[0, :]
    ylo = boxv[1, :]
    xhi = boxv[2, :]
    yhi = boxv[3, :]
    lanes = lax.iota(jnp.int32, 16)
    lane0 = lanes == 0

    def process_rows(rlo, rhi, cnt_vec):
        @plsc.parallel_loop(rlo, rhi, step=1, unroll=2, carry=cnt_vec)
        def row_body(ri, cv):
            for k in range(VPR):
                xv = xbuf[ri, pl.ds(k * 16, 16)]
                yv = ybuf[ri, pl.ds(k * 16, 16)]
                m = (xv >= xlo) & (xv <= xhi) & (yv >= ylo) & (yv <= yhi)
                cv = cv + plsc.all_reduce_population_count(m)
            plsc.store_scatter(rowbuf, [lanes * 0 + ri], cv, mask=lane0)
            return cv

        return row_body

    def chunk_body(ci, cnt_vec):
        r0 = (c0 + ci) * ROWS_C
        pltpu.sync_copy(x_hbm.at[pl.ds(r0, ROWS_C), :], xbuf)
        pltpu.sync_copy(y_hbm.at[pl.ds(r0, ROWS_C), :], ybuf)
        cnt_vec = process_rows(0, ROWS_C, cnt_vec)
        pltpu.sync_copy(rowbuf.at[pl.ds(0, ROWS_C)],
                        rowincl_hbm.at[pl.ds(r0, ROWS_C)])
        return cnt_vec

    cnt_vec = lax.fori_loop(0, nch, chunk_body, jnp.zeros((16,), jnp.int32))
    tr0 = NCH_FULL * ROWS_C
    pltpu.sync_copy(x_hbm.at[pl.ds(tr0, TAIL_ROWS), :],
                    xbuf.at[pl.ds(0, TAIL_ROWS), :])
    pltpu.sync_copy(y_hbm.at[pl.ds(tr0, TAIL_ROWS), :],
                    ybuf.at[pl.ds(0, TAIL_ROWS), :])
    trlo = jnp.where(wid == NW - 1, 0, TAIL_ROWS)
    cnt_vec = process_rows(trlo, TAIL_ROWS, cnt_vec)

    @pl.when(wid == NW - 1)
    def _():
        pltpu.sync_copy(rowbuf.at[pl.ds(0, TAIL_ROWS)],
                        rowincl_hbm.at[pl.ds(tr0, TAIL_ROWS)])

    cntv[...] = cnt_vec
    pltpu.sync_copy(cntv, counts_hbm.at[wid])


_STAGE_A = pl.kernel(
    _stage_a_body,
    out_type=(
        jax.ShapeDtypeStruct((NW, 16), jnp.int32),
        jax.ShapeDtypeStruct((NROWS,), jnp.int32),
    ),
    mesh=_mesh,
    scratch_types=(
        pltpu.VMEM((ROWS_C, 128), jnp.float32),
        pltpu.VMEM((ROWS_C, 128), jnp.float32),
        pltpu.VMEM((ROWS_C,), jnp.int32),
        pltpu.VMEM((4, 16), jnp.float32),
        pltpu.VMEM((16,), jnp.int32),
    ),
    compiler_params=pltpu.CompilerParams(needs_layout_passes=False),
)


def _worker_of_row(row):
    chunk = row >> 6
    w_lo = chunk >> 3
    w_hi = 20 + (((chunk - 160) * 9363) >> 16)
    w = jnp.where(chunk < 160, w_lo, w_hi)
    return jnp.minimum(w, NW - 1)


def _stage_b_body(x_hbm, y_hbm, seq_hbm, rowincl_hbm, counts_hbm, pn_hbm,
                  prm_hbm, out_hbm,
                  cntbuf, basebuf, rowbuf, pnbuf, rowsb, lrb, origbuf,
                  xrows, yrows, valbuf, prmbuf, dsem):
    wid = lax.axis_index("c") * 16 + lax.axis_index("s")
    qstart = jnp.minimum(wid * QPW, NQ - QPW)
    pltpu.sync_copy(counts_hbm, cntbuf)
    pltpu.sync_copy(rowincl_hbm, rowbuf)
    pltpu.sync_copy(pn_hbm.at[pl.ds(qstart, QPW)], pnbuf)
    pltpu.sync_copy(prm_hbm, prmbuf)
    lanes = lax.iota(jnp.int32, 16)
    zeros = jnp.zeros((16,), jnp.int32)
    c_lo = plsc.load_gather(cntbuf, [lanes, zeros])
    c_hi = plsc.load_gather(cntbuf, [lanes + 16, zeros])
    incl_lo = plsc.cumsum(c_lo)
    t_lo = jnp.sum(c_lo)
    incl_hi = plsc.cumsum(c_hi) + t_lo
    basebuf[pl.ds(0, 16)] = incl_lo - c_lo
    basebuf[pl.ds(16, 16)] = incl_hi - c_hi

    def gincl(row):
        loc = plsc.load_gather(rowbuf, [row])
        return loc + plsc.load_gather(basebuf, [_worker_of_row(row)])

    def rank_body(qv, carry):
        r = pnbuf[pl.ds(qv * 16, 16)]
        row = jnp.zeros((16,), jnp.int32)
        for bit in (8192, 4096, 2048, 1024, 512, 256, 128, 64, 32, 16,
                    8, 4, 2, 1):
            probe = jnp.minimum(row + (bit - 1), NROWS - 1)
            row = row + jnp.where(r >= gincl(probe), bit, 0)
        prev = jnp.maximum(row - 1, 0)
        ex = jnp.where(row > 0, gincl(prev), 0)
        lr = r - ex
        rowsb[pl.ds(qv * 16, 16)] = row
        lrb[pl.ds(qv * 16, 16)] = lr
        pltpu.async_copy(x_hbm.at[row], xrows.at[pl.ds(qv * 16, 16), :], dsem)
        pltpu.async_copy(y_hbm.at[row], yrows.at[pl.ds(qv * 16, 16), :], dsem)
        return carry

    lax.fori_loop(0, QV, rank_body, 0)
    pltpu.make_async_copy(x_hbm.at[pl.ds(0, QPW), :], xrows, dsem).wait()
    pltpu.make_async_copy(y_hbm.at[pl.ds(0, QPW), :], yrows, dsem).wait()

    xlo = prmbuf[0, :]
    ylo = prmbuf[1, :]
    xhi = prmbuf[2, :]
    yhi = prmbuf[3, :]

    def sel_body(qv, carry):
        qid = qv * 16 + lanes
        lr = lrb[pl.ds(qv * 16, 16)]
        row = rowsb[pl.ds(qv * 16, 16)]

        def col_body(c, rp):
            run, pos = rp
            cc = lanes * 0 + c
            xv = plsc.load_gather(xrows, [qid, cc])
            yv = plsc.load_gather(yrows, [qid, cc])
            m = (xv >= xlo) & (xv <= xhi) & (yv >= ylo) & (yv <= yhi)
            hit = m & (run == lr)
            pos = jnp.where(hit, c, pos)
            return run + jnp.where(m, 1, 0), pos

        _, pos = lax.fori_loop(0, 128, col_body, (zeros, zeros))
        origbuf[pl.ds(qv * 16, 16)] = row * 128 + pos
        return carry

    lax.fori_loop(0, QV, sel_body, 0)

    def gat_body(j, carry):
        c = j // QV
        rr = j - c * QV
        ov = origbuf[pl.ds(rr * 16, 16)]
        cidx = ov + c * L_EV
        pltpu.async_copy(seq_hbm.at[cidx], valbuf.at[pl.ds(j * 16, 16)], dsem)
        return carry

    lax.fori_loop(0, 5 * QV, gat_body, 0)
    pltpu.make_async_copy(seq_hbm.at[pl.ds(0, 5 * QPW)], valbuf, dsem).wait()

    for ch in range(2):
        lov = prmbuf[ch, :]
        dv = prmbuf[4 + ch, :]
        for rr in range(QV):
            j = ch * QV + rr
            v = valbuf[pl.ds(j * 16, 16)]
            valbuf[pl.ds(j * 16, 16)] = (v - lov) / dv

    for c in range(5):
        pltpu.sync_copy(valbuf.at[pl.ds(c * QPW, QPW)],
                        out_hbm.at[pl.ds(c * NQ + qstart, QPW)])


_STAGE_B = pl.kernel(
    _stage_b_body,
    out_type=jax.ShapeDtypeStruct((5 * NQ,), jnp.float32),
    mesh=_mesh,
    scratch_types=(
        pltpu.VMEM((NW, 16), jnp.int32),
        pltpu.VMEM((NW,), jnp.int32),
        pltpu.VMEM((NROWS,), jnp.int32),
        pltpu.VMEM((QPW,), jnp.int32),
        pltpu.VMEM((QPW,), jnp.int32),
        pltpu.VMEM((QPW,), jnp.int32),
        pltpu.VMEM((QPW,), jnp.int32),
        pltpu.VMEM((QPW, 128), jnp.float32),
        pltpu.VMEM((QPW, 128), jnp.float32),
        pltpu.VMEM((5 * QPW,), jnp.float32),
        pltpu.VMEM((6, 16), jnp.float32),
        pltpu.SemaphoreType.DMA,
    ),
    compiler_params=pltpu.CompilerParams(needs_layout_passes=False),
)


def kernel(Seq, previous_pred):
    lo = jnp.clip(previous_pred[:2] - previous_pred[2:] / 2 - 0.25, 0.0, 1.0)
    hi = jnp.clip(lo + previous_pred[2:] + 0.5, 0.0, 1.0)
    lo = jnp.clip(hi - previous_pred[2:] - 0.5, 0.0, 1.0)
    seq_flat = jnp.reshape(Seq, (-1,))
    x2d = jnp.reshape(Seq[0, 0], (-1, 128))
    y2d = jnp.reshape(Seq[0, 1], (-1, 128))
    box = jnp.stack([lo[0], lo[1], hi[0], hi[1]])
    box_b = jnp.broadcast_to(box[:, None], (4, 16))
    counts, rowincl = _STAGE_A(x2d, y2d, box_b)
    n_total = jnp.sum(counts[:, 0])
    pn = jax.random.randint(jax.random.key(1), (NQ,), 0, n_total)
    d0 = hi[0] - lo[0] + 1e-6
    d1 = hi[1] - lo[1] + 1e-6
    prm = jnp.stack([lo[0], lo[1], hi[0], hi[1], d0, d1])
    prm_b = jnp.broadcast_to(prm[:, None], (6, 16))
    out = _STAGE_B(x2d, y2d, seq_flat, rowincl, counts,
                   pn.astype(jnp.int32), prm_b)
    return jnp.reshape(out, (1, 5, NQ)), lo, hi
